# baseline (speedup 1.0000x reference)
import functools

import jax
import jax.numpy as jnp
from jax import lax
from jax.experimental import pallas as pl
from jax.experimental.pallas import tpu as pltpu
from jax.experimental.pallas import tpu_sc as plsc

_NC = 2
_NS = 16
_NW = _NC * _NS
_LANES = 128
_CH = 128


def _table_body(sfeat_ref, adj_ref, w1t_ref, b1_ref, w2t_ref, b2_ref, out_ref):
    h = jnp.dot(sfeat_ref[...], w1t_ref[...], preferred_element_type=jnp.float32)
    h = jnp.maximum(h + b1_ref[...], 0.0)
    h = h * adj_ref[...]
    y = jnp.dot(h, w2t_ref[...], preferred_element_type=jnp.float32)
    y = jnp.maximum(y + b2_ref[...], 0.0)
    m = jnp.max(y, axis=1, keepdims=True)
    lse = jnp.log(jnp.sum(jnp.exp(y - m), axis=1, keepdims=True)) + m
    ls = y - lse
    out_ref[...] = jnp.concatenate([ls, ls], axis=1)


def _compute_table(Sfeatures, adj, W1, b1, W2, b2):
    n = Sfeatures.shape[0]
    hid = W1.shape[0]
    out_f = W2.shape[0]
    return pl.pallas_call(
        _table_body,
        out_shape=jax.ShapeDtypeStruct((n, _LANES), jnp.float32),
    )(Sfeatures, adj, W1.T, b1.reshape(1, hid), W2.T, b2.reshape(1, out_f))


def _gather_rows_t(table, idx, out_f):
    e = idx.shape[0]
    n = table.shape[0]
    n_chunks = e // _CH
    mesh = plsc.VectorSubcoreMesh(
        core_axis_name="c", subcore_axis_name="s", num_cores=_NC, num_subcores=_NS
    )

    @functools.partial(
        pl.kernel,
        mesh=mesh,
        compiler_params=pltpu.CompilerParams(needs_layout_passes=False),
        out_type=jax.ShapeDtypeStruct((out_f, e), jnp.float32),
        scratch_types=[
            pltpu.VMEM_SHARED((n, _LANES), jnp.float32),
            pltpu.VMEM((_CH,), jnp.int32),
            pltpu.VMEM((_CH,), jnp.int32),
            pltpu.VMEM((_CH, _LANES), jnp.float32),
            pltpu.VMEM((_CH, _LANES), jnp.float32),
            pltpu.VMEM((out_f, _CH), jnp.float32),
            pltpu.VMEM((out_f, _CH), jnp.float32),
            pltpu.SemaphoreType.DMA,
            pltpu.SemaphoreType.DMA,
            pltpu.SemaphoreType.DMA,
            pltpu.SemaphoreType.DMA,
            pltpu.SemaphoreType.DMA,
            pltpu.SemaphoreType.DMA,
        ],
    )
    def gather_k(
        table_hbm, idx_hbm, out_hbm, table_sh,
        idx0, idx1, rows0, rows1, rt0, rt1, g0, g1, w0, w1, i0, i1,
    ):
        idx_v = [idx0, idx1]
        rows_v = [rows0, rows1]
        rt_v = [rt0, rt1]
        gsem = [g0, g1]
        wsem = [w0, w1]
        isem = [i0, i1]

        cid = lax.axis_index("c")
        sid = lax.axis_index("s")
        wid = sid * _NC + cid

        @pl.when(sid == 0)
        def _():
            pltpu.sync_copy(table_hbm, table_sh)

        plsc.subcore_barrier()

        n_mine = (n_chunks - wid + _NW - 1) // _NW
        n_groups = (max((n_chunks + _NW - 1) // _NW, 2) + 1) // 2
        lane_iota = lax.iota(jnp.int32, 16)

        def chunk_off(t):
            return (wid + t * _NW) * _CH

        def start_idx(q, t):
            pltpu.async_copy(idx_hbm.at[pl.ds(chunk_off(t), _CH)], idx_v[q], isem[q])

        def wait_idx(q, t):
            pltpu.make_async_copy(
                idx_hbm.at[pl.ds(chunk_off(t), _CH)], idx_v[q], isem[q]
            ).wait()

        def start_gather(q):
            pltpu.async_copy(table_sh.at[idx_v[q]], rows_v[q], gsem[q])

        def transpose_chunk(p):
            for fg in range(out_f // 16):
                fvec = lane_iota + 16 * fg
                for eg in range(_CH // 8):
                    vs = [
                        rows_v[p][8 * eg + i, pl.ds(16 * fg, 16)]
                        for i in range(8)
                    ]
                    for i in range(8):
                        plsc.store_scatter(
                            rt_v[p],
                            [fvec, jnp.full((16,), 8 * eg + i, jnp.int32)],
                            vs[i],
                        )

        start_idx(0, 0)
        start_idx(1, 1)
        wait_idx(0, 0)
        start_gather(0)

        def group_body(g, carry):
            for p in (0, 1):
                t = 2 * g + p

                @pl.when(t < n_mine)
                def _():
                    @pl.when(t >= 2)
                    def _():
                        pltpu.make_async_copy(
                            rt_v[p],
                            out_hbm.at[:, pl.ds(chunk_off(t - 2), _CH)],
                            wsem[p],
                        ).wait()

                    pltpu.make_async_copy(
                        table_sh.at[idx_v[p]], rows_v[p], gsem[p]
                    ).wait()

                    @pl.when(t + 2 < n_mine)
                    def _():
                        start_idx(p, t + 2)

                    @pl.when(t + 1 < n_mine)
                    def _():
                        wait_idx(1 - p, t + 1)
                        start_gather(1 - p)

                    transpose_chunk(p)
                    pltpu.async_copy(
                        rt_v[p], out_hbm.at[:, pl.ds(chunk_off(t), _CH)], wsem[p]
                    )

            return carry

        lax.fori_loop(0, n_groups, group_body, 0)

        for p in (0, 1):
            pltpu.make_async_copy(
                rt_v[p], out_hbm.at[:, pl.ds(chunk_off(0), _CH)], wsem[p]
            ).wait()

    return gather_k(table, idx)


def kernel(source_index, adj, Sfeatures, W1, b1, W2, b2):
    table = _compute_table(Sfeatures, adj, W1, b1, W2, b2)
    out_t = _gather_rows_t(table, source_index, W2.shape[0])
    return jnp.transpose(out_t)


# speedup vs baseline: 2.3384x; 2.3384x over previous
import functools

import jax
import jax.numpy as jnp
from jax import lax
from jax.experimental import pallas as pl
from jax.experimental.pallas import tpu as pltpu
from jax.experimental.pallas import tpu_sc as plsc

_NC = 2
_NS = 16
_NW = _NC * _NS
_LANES = 128
_CH = 128


def _table_body(sfeat_ref, adj_ref, w1t_ref, b1_ref, w2t_ref, b2_ref, out_ref):
    h = jnp.dot(sfeat_ref[...], w1t_ref[...], preferred_element_type=jnp.float32)
    h = jnp.maximum(h + b1_ref[...], 0.0)
    h = h * adj_ref[...]
    y = jnp.dot(h, w2t_ref[...], preferred_element_type=jnp.float32)
    y = jnp.maximum(y + b2_ref[...], 0.0)
    m = jnp.max(y, axis=1, keepdims=True)
    lse = jnp.log(jnp.sum(jnp.exp(y - m), axis=1, keepdims=True)) + m
    ls = y - lse
    out_ref[...] = jnp.concatenate([ls, ls], axis=1)


def _compute_table(Sfeatures, adj, W1, b1, W2, b2):
    n = Sfeatures.shape[0]
    hid = W1.shape[0]
    out_f = W2.shape[0]
    return pl.pallas_call(
        _table_body,
        out_shape=jax.ShapeDtypeStruct((n, _LANES), jnp.float32),
    )(Sfeatures, adj, W1.T, b1.reshape(1, hid), W2.T, b2.reshape(1, out_f))


def _gather_rows_t(table, idx, out_f):
    e = idx.shape[0]
    n = table.shape[0]
    n_chunks = e // _CH
    mesh = plsc.VectorSubcoreMesh(
        core_axis_name="c", subcore_axis_name="s", num_cores=_NC, num_subcores=_NS
    )

    @functools.partial(
        pl.kernel,
        mesh=mesh,
        compiler_params=pltpu.CompilerParams(needs_layout_passes=False),
        out_type=jax.ShapeDtypeStruct((out_f, e), jnp.float32),
        scratch_types=[
            pltpu.VMEM_SHARED((n, _LANES), jnp.float32),
            pltpu.VMEM((_CH,), jnp.int32),
            pltpu.VMEM((_CH,), jnp.int32),
            pltpu.VMEM((_CH, _LANES), jnp.float32),
            pltpu.VMEM((_CH, _LANES), jnp.float32),
            pltpu.VMEM((out_f, _CH), jnp.float32),
            pltpu.SemaphoreType.DMA,
            pltpu.SemaphoreType.DMA,
            pltpu.SemaphoreType.DMA,
            pltpu.SemaphoreType.DMA,
            pltpu.SemaphoreType.DMA,
            pltpu.SemaphoreType.DMA,
        ],
    )
    def gather_k(
        table_hbm, idx_hbm, out_hbm, table_sh,
        idx0, idx1, rows0, rows1, rt0, g0, g1, w0, w1, i0, i1,
    ):
        idx_v = [idx0, idx1]
        rows_v = [rows0, rows1]
        rt_v = [rt0, rt0]
        gsem = [g0, g1]
        wsem = [w0, w1]
        isem = [i0, i1]

        cid = lax.axis_index("c")
        sid = lax.axis_index("s")
        wid = sid * _NC + cid

        @pl.when(sid == 0)
        def _():
            pltpu.sync_copy(table_hbm, table_sh)

        plsc.subcore_barrier()

        n_mine = (n_chunks - wid + _NW - 1) // _NW
        n_groups = (max((n_chunks + _NW - 1) // _NW, 2) + 1) // 2
        lane_iota = lax.iota(jnp.int32, 16)

        def chunk_off(t):
            return (wid + t * _NW) * _CH

        def start_idx(q, t):
            pltpu.async_copy(idx_hbm.at[pl.ds(chunk_off(t), _CH)], idx_v[q], isem[q])

        def wait_idx(q, t):
            pltpu.make_async_copy(
                idx_hbm.at[pl.ds(chunk_off(t), _CH)], idx_v[q], isem[q]
            ).wait()

        def start_gather(q):
            pltpu.async_copy(table_sh.at[idx_v[q]], rows_v[q], gsem[q])

        def transpose_chunk(p):
            def eb_body(eb, carry):
                evec = lane_iota + 16 * eb
                for fb in range(out_f // 16):
                    for dg in range(4):
                        fperms = [
                            ((lane_iota + (4 * dg + i)) & 15) + 16 * fb
                            for i in range(4)
                        ]
                        vs = [
                            plsc.load_gather(rows_v[p], [evec, fperms[i]])
                            for i in range(4)
                        ]
                        for i in range(4):
                            plsc.store_scatter(
                                rt_v[p], [fperms[i], evec], vs[i]
                            )
                return carry

            lax.fori_loop(0, _CH // 16, eb_body, 0)

        start_idx(0, 0)
        start_idx(1, 1)
        wait_idx(0, 0)
        start_gather(0)

        def group_body(g, carry):
            for p in (0, 1):
                t = 2 * g + p

                @pl.when(t < n_mine)
                def _():
                    @pl.when(t >= 1)
                    def _():
                        pltpu.make_async_copy(
                            rt_v[p],
                            out_hbm.at[:, pl.ds(chunk_off(t - 1), _CH)],
                            wsem[1 - p],
                        ).wait()

                    pltpu.make_async_copy(
                        table_sh.at[idx_v[p]], rows_v[p], gsem[p]
                    ).wait()

                    @pl.when(t + 2 < n_mine)
                    def _():
                        start_idx(p, t + 2)

                    @pl.when(t + 1 < n_mine)
                    def _():
                        wait_idx(1 - p, t + 1)
                        start_gather(1 - p)

                    transpose_chunk(p)
                    pltpu.async_copy(
                        rt_v[p], out_hbm.at[:, pl.ds(chunk_off(t), _CH)], wsem[p]
                    )

            return carry

        lax.fori_loop(0, n_groups, group_body, 0)

        @pl.when(n_mine % 2 == 1)
        def _():
            pltpu.make_async_copy(
                rt_v[0], out_hbm.at[:, pl.ds(chunk_off(0), _CH)], wsem[0]
            ).wait()

        @pl.when(n_mine % 2 == 0)
        def _():
            pltpu.make_async_copy(
                rt_v[0], out_hbm.at[:, pl.ds(chunk_off(0), _CH)], wsem[1]
            ).wait()

    return gather_k(table, idx)


def kernel(source_index, adj, Sfeatures, W1, b1, W2, b2):
    table = _compute_table(Sfeatures, adj, W1, b1, W2, b2)
    out_t = _gather_rows_t(table, source_index, W2.shape[0])
    return jnp.transpose(out_t)
